# baseline (speedup 1.0000x reference)
import dataclasses
import functools

import jax
import jax.numpy as jnp
from jax import lax
from jax.experimental import pallas as pl
from jax.experimental.pallas import tpu as pltpu
from jax.experimental.pallas import tpu_sc as plsc

B = 16
IN_LEN = 50000
OUT_LEN = 5000
NNZ = 1600000
NCORES = 2
LANES = 16
NQ = 4
QLEN = 1280
OUT_PAD = NQ * QLEN
SBLK = 8192
LSTRIDE = SBLK // LANES
NBLK = -(-NNZ // SBLK)
NNZ_PAD = NBLK * SBLK
CHUNK = 2048
UNROLL = 4


def _body(x_hbm, pk_hbm, w_hbm, bias_hbm, off_hbm, out_hbm,
          xb0, xb1, acc0, acc1, pbuf0, wbuf0, pbuf1, wbuf1, offv, bbuf,
          sem_a, sem_b):
    c_idx = lax.axis_index("c")
    s_idx = lax.axis_index("s")
    wid = s_idx * NCORES + c_idx
    bp = wid % (B // 2)
    q = wid // (B // 2)

    pltpu.sync_copy(x_hbm.at[2 * bp], xb0)
    pltpu.sync_copy(x_hbm.at[2 * bp + 1], xb1)
    pltpu.sync_copy(off_hbm, offv)

    iot = lax.iota(jnp.int32, LANES)
    iot_s = iot * LSTRIDE
    ov = offv[...]
    n_lo = jnp.sum(jnp.where(iot == q, ov, 0))
    n_hi = jnp.sum(jnp.where(iot == q + 1, ov, 0))

    c0 = q * QLEN
    pltpu.sync_copy(bias_hbm.at[pl.ds(c0, QLEN)], bbuf)

    @pl.loop(0, QLEN, step=LANES)
    def _init(j):
        bv = bbuf[pl.ds(j, LANES)]
        acc0[pl.ds(j, LANES)] = bv
        acc1[pl.ds(j, LANES)] = bv

    lo_al = n_lo & ~(SBLK - 1)
    hi_end = (n_hi + SBLK - 1) & ~(SBLK - 1)
    nchunks = (hi_end - lo_al) // CHUNK

    def copies(k, pb, wb, sem):
        base = pl.multiple_of(
            jnp.minimum(lo_al + k * CHUNK, NNZ_PAD - CHUNK), CHUNK)
        return (
            pltpu.make_async_copy(pk_hbm.at[pl.ds(base, CHUNK)], pb, sem),
            pltpu.make_async_copy(w_hbm.at[pl.ds(base, CHUNK)], wb, sem),
        )

    def issue(k, pb, wb, sem):
        for c in copies(k, pb, wb, sem):
            c.start()

    def drain(k, pb, wb, sem):
        for c in copies(k, pb, wb, sem):
            c.wait()

    def compute(k, pb, wb):
        start = lo_al + k * CHUNK
        blk0 = start & ~(SBLK - 1)
        interior = (blk0 >= n_lo) & (blk0 + SBLK <= n_hi)

        @pl.when(interior)
        def _fast():
            @plsc.parallel_loop(0, CHUNK, step=LANES, unroll=UNROLL)
            def _grp(j):
                sl = pl.ds(j, LANES)
                pk = pb[sl]
                r = pk & 0xFFFF
                cv = lax.shift_right_logical(pk, 16)
                wv = wb[sl]
                plsc.addupdate_scatter(
                    acc0, [cv], plsc.load_gather(xb0, [r]) * wv)
                plsc.addupdate_scatter(
                    acc1, [cv], plsc.load_gather(xb1, [r]) * wv)

        @pl.when(jnp.logical_not(interior))
        def _masked():
            gbase = blk0 + ((start & (SBLK - 1)) >> 4)

            @plsc.parallel_loop(0, CHUNK, step=LANES, unroll=UNROLL)
            def _grp(j):
                g = (gbase + (j >> 4)) + iot_s
                m = (g >= n_lo) & (g < n_hi)
                sl = pl.ds(j, LANES)
                pk = pb[sl]
                r = pk & 0xFFFF
                cv = lax.shift_right_logical(pk, 16)
                wv = wb[sl]
                plsc.addupdate_scatter(
                    acc0, [cv], plsc.load_gather(xb0, [r]) * wv, mask=m)
                plsc.addupdate_scatter(
                    acc1, [cv], plsc.load_gather(xb1, [r]) * wv, mask=m)

    issue(0, pbuf0, wbuf0, sem_a)
    npairs = (nchunks + 1) // 2

    def pair(p, carry):
        k0 = 2 * p
        drain(k0, pbuf0, wbuf0, sem_a)
        issue(k0 + 1, pbuf1, wbuf1, sem_b)
        compute(k0, pbuf0, wbuf0)
        drain(k0 + 1, pbuf1, wbuf1, sem_b)
        issue(k0 + 2, pbuf0, wbuf0, sem_a)
        compute(k0 + 1, pbuf1, wbuf1)
        return carry

    lax.fori_loop(0, npairs, pair, 0)
    drain(2 * npairs, pbuf0, wbuf0, sem_a)

    pltpu.sync_copy(acc0, out_hbm.at[2 * bp, pl.ds(c0, QLEN)])
    pltpu.sync_copy(acc1, out_hbm.at[2 * bp + 1, pl.ds(c0, QLEN)])


def _stripe(a):
    a = jnp.pad(a, (0, NNZ_PAD - NNZ))
    return a.reshape(NBLK, LANES, LSTRIDE).transpose(0, 2, 1).reshape(-1)


@jax.jit
def kernel(x, mask_rows, mask_cols, kernel, bias):
    x2 = x.reshape(B, IN_LEN)
    bias_pad = jnp.pad(bias[:, 0], (0, OUT_PAD - OUT_LEN))
    qb = jnp.searchsorted(
        mask_cols, jnp.array([QLEN, 2 * QLEN, 3 * QLEN], jnp.int32)
    ).astype(jnp.int32)
    off = jnp.zeros((LANES,), jnp.int32)
    off = off.at[1:4].set(qb)
    off = off.at[4:].set(NNZ)

    local_col = mask_cols - (mask_cols // QLEN) * QLEN
    packed = jnp.bitwise_or(jnp.left_shift(local_col, 16), mask_rows)
    packed, w_in = lax.optimization_barrier((packed, kernel))
    pk_p = _stripe(packed)
    w_p = _stripe(w_in)

    mesh = plsc.VectorSubcoreMesh(core_axis_name="c", subcore_axis_name="s")
    cp = pltpu.CompilerParams()
    if "needs_layout_passes" in pltpu.CompilerParams.__dataclass_fields__:
        cp = dataclasses.replace(cp, needs_layout_passes=False)
    run = functools.partial(
        pl.kernel,
        compiler_params=cp,
        out_type=jax.ShapeDtypeStruct((B, OUT_PAD), jnp.float32),
        mesh=mesh,
        scratch_types=[
            pltpu.VMEM((IN_LEN,), jnp.float32),
            pltpu.VMEM((IN_LEN,), jnp.float32),
            pltpu.VMEM((QLEN,), jnp.float32),
            pltpu.VMEM((QLEN,), jnp.float32),
            pltpu.VMEM((CHUNK,), jnp.int32),
            pltpu.VMEM((CHUNK,), jnp.float32),
            pltpu.VMEM((CHUNK,), jnp.int32),
            pltpu.VMEM((CHUNK,), jnp.float32),
            pltpu.VMEM((LANES,), jnp.int32),
            pltpu.VMEM((QLEN,), jnp.float32),
            pltpu.SemaphoreType.DMA,
            pltpu.SemaphoreType.DMA,
        ],
    )(_body)
    outp = run(x2, pk_p, w_p, bias_pad, off)
    return outp[:, :OUT_LEN].reshape(B, OUT_LEN, 1)


# speedup vs baseline: 1.2444x; 1.2444x over previous
import dataclasses
import functools

import jax
import jax.numpy as jnp
from jax import lax
from jax.experimental import pallas as pl
from jax.experimental.pallas import tpu as pltpu
from jax.experimental.pallas import tpu_sc as plsc

B = 16
IN_LEN = 50000
OUT_LEN = 5000
NNZ = 1600000
NCORES = 2
LANES = 16
NE = 8
ELEN = 640
OUT_PAD = NE * ELEN
SBLK = 8192
LSTRIDE = SBLK // LANES
NBLK = -(-NNZ // SBLK)
NNZ_PAD = NBLK * SBLK
CHUNK = 2048
UNROLL = 4


def _body(xp_hbm, pk_hbm, w_hbm, bias_hbm, off_hbm, out_hbm,
          xq0, xq1, acc0, acc1, acc2, acc3, pbuf0, wbuf0, pbuf1, wbuf1,
          offv, bbuf, sem_a, sem_b):
    c_idx = lax.axis_index("c")
    s_idx = lax.axis_index("s")
    wid = s_idx * NCORES + c_idx
    bq = wid % (B // 4)
    e = wid // (B // 4)

    pltpu.sync_copy(xp_hbm.at[2 * bq], xq0)
    pltpu.sync_copy(xp_hbm.at[2 * bq + 1], xq1)
    pltpu.sync_copy(off_hbm, offv)

    iot = lax.iota(jnp.int32, LANES)
    iot_s = iot * LSTRIDE
    ov = offv[...]
    n_lo = jnp.sum(jnp.where(iot == e, ov, 0))
    n_hi = jnp.sum(jnp.where(iot == e + 1, ov, 0))

    c0 = e * ELEN
    pltpu.sync_copy(bias_hbm.at[pl.ds(c0, ELEN)], bbuf)

    @pl.loop(0, ELEN, step=LANES)
    def _init(j):
        bv = bbuf[pl.ds(j, LANES)]
        acc0[pl.ds(j, LANES)] = bv
        acc1[pl.ds(j, LANES)] = bv
        acc2[pl.ds(j, LANES)] = bv
        acc3[pl.ds(j, LANES)] = bv

    lo_al = n_lo & ~(SBLK - 1)
    hi_end = (n_hi + SBLK - 1) & ~(SBLK - 1)
    nchunks = (hi_end - lo_al) // CHUNK

    def copies(k, pb, wb, sem):
        base = pl.multiple_of(
            jnp.minimum(lo_al + k * CHUNK, NNZ_PAD - CHUNK), CHUNK)
        return (
            pltpu.make_async_copy(pk_hbm.at[pl.ds(base, CHUNK)], pb, sem),
            pltpu.make_async_copy(w_hbm.at[pl.ds(base, CHUNK)], wb, sem),
        )

    def issue(k, pb, wb, sem):
        for c in copies(k, pb, wb, sem):
            c.start()

    def drain(k, pb, wb, sem):
        for c in copies(k, pb, wb, sem):
            c.wait()

    def macs(pb, wb, j, m):
        sl = pl.ds(j, LANES)
        pk = pb[sl]
        r = pk & 0xFFFF
        cv = jnp.right_shift(pk, 16)
        wv = wb[sl]
        x01 = plsc.load_gather(xq0, [r])
        x23 = plsc.load_gather(xq1, [r])
        xe0 = plsc.bitcast(jnp.left_shift(x01, 16), jnp.float32)
        xe1 = plsc.bitcast(x01 & jnp.int32(-65536), jnp.float32)
        xe2 = plsc.bitcast(jnp.left_shift(x23, 16), jnp.float32)
        xe3 = plsc.bitcast(x23 & jnp.int32(-65536), jnp.float32)
        plsc.addupdate_scatter(acc0, [cv], xe0 * wv, mask=m)
        plsc.addupdate_scatter(acc1, [cv], xe1 * wv, mask=m)
        plsc.addupdate_scatter(acc2, [cv], xe2 * wv, mask=m)
        plsc.addupdate_scatter(acc3, [cv], xe3 * wv, mask=m)

    def compute(k, pb, wb):
        start = lo_al + k * CHUNK
        blk0 = start & ~(SBLK - 1)
        interior = (blk0 >= n_lo) & (blk0 + SBLK <= n_hi)

        @pl.when(interior)
        def _fast():
            @plsc.parallel_loop(0, CHUNK, step=LANES, unroll=UNROLL)
            def _grp(j):
                macs(pb, wb, j, None)

        @pl.when(jnp.logical_not(interior))
        def _masked():
            gbase = blk0 + ((start & (SBLK - 1)) >> 4)

            @plsc.parallel_loop(0, CHUNK, step=LANES, unroll=UNROLL)
            def _grp(j):
                g = (gbase + (j >> 4)) + iot_s
                m = (g >= n_lo) & (g < n_hi)
                macs(pb, wb, j, m)

    issue(0, pbuf0, wbuf0, sem_a)
    npairs = (nchunks + 1) // 2

    def pair(p, carry):
        k0 = 2 * p
        drain(k0, pbuf0, wbuf0, sem_a)
        issue(k0 + 1, pbuf1, wbuf1, sem_b)
        compute(k0, pbuf0, wbuf0)
        drain(k0 + 1, pbuf1, wbuf1, sem_b)
        issue(k0 + 2, pbuf0, wbuf0, sem_a)
        compute(k0 + 1, pbuf1, wbuf1)
        return carry

    lax.fori_loop(0, npairs, pair, 0)
    drain(2 * npairs, pbuf0, wbuf0, sem_a)

    pltpu.sync_copy(acc0, out_hbm.at[4 * bq + 0, pl.ds(c0, ELEN)])
    pltpu.sync_copy(acc1, out_hbm.at[4 * bq + 1, pl.ds(c0, ELEN)])
    pltpu.sync_copy(acc2, out_hbm.at[4 * bq + 2, pl.ds(c0, ELEN)])
    pltpu.sync_copy(acc3, out_hbm.at[4 * bq + 3, pl.ds(c0, ELEN)])


def _stripe(a):
    a = jnp.pad(a, (0, NNZ_PAD - NNZ))
    return a.reshape(NBLK, LANES, LSTRIDE).transpose(0, 2, 1).reshape(-1)


@jax.jit
def kernel(x, mask_rows, mask_cols, kernel, bias):
    x2 = x.reshape(B, IN_LEN)
    xbf = lax.bitcast_convert_type(
        x2.astype(jnp.bfloat16), jnp.uint16).astype(jnp.uint32)
    xp = lax.bitcast_convert_type(
        xbf[0::2] | (xbf[1::2] << 16), jnp.int32)

    bias_pad = jnp.pad(bias[:, 0], (0, OUT_PAD - OUT_LEN))
    eb = jnp.searchsorted(
        mask_cols, jnp.arange(ELEN, OUT_LEN, ELEN, dtype=jnp.int32)
    ).astype(jnp.int32)
    off = jnp.zeros((LANES,), jnp.int32)
    off = off.at[1:NE].set(eb)
    off = off.at[NE:].set(NNZ)

    local_col = mask_cols - (mask_cols // ELEN) * ELEN
    packed = jnp.bitwise_or(jnp.left_shift(local_col, 16), mask_rows)
    pk_p = _stripe(packed)
    w_p = _stripe(kernel)

    mesh = plsc.VectorSubcoreMesh(core_axis_name="c", subcore_axis_name="s")
    cp = pltpu.CompilerParams()
    if "needs_layout_passes" in pltpu.CompilerParams.__dataclass_fields__:
        cp = dataclasses.replace(cp, needs_layout_passes=False)
    run = functools.partial(
        pl.kernel,
        compiler_params=cp,
        out_type=jax.ShapeDtypeStruct((B, OUT_PAD), jnp.float32),
        mesh=mesh,
        scratch_types=[
            pltpu.VMEM((IN_LEN,), jnp.int32),
            pltpu.VMEM((IN_LEN,), jnp.int32),
            pltpu.VMEM((ELEN,), jnp.float32),
            pltpu.VMEM((ELEN,), jnp.float32),
            pltpu.VMEM((ELEN,), jnp.float32),
            pltpu.VMEM((ELEN,), jnp.float32),
            pltpu.VMEM((CHUNK,), jnp.int32),
            pltpu.VMEM((CHUNK,), jnp.float32),
            pltpu.VMEM((CHUNK,), jnp.int32),
            pltpu.VMEM((CHUNK,), jnp.float32),
            pltpu.VMEM((LANES,), jnp.int32),
            pltpu.VMEM((ELEN,), jnp.float32),
            pltpu.SemaphoreType.DMA,
            pltpu.SemaphoreType.DMA,
        ],
    )(_body)
    outp = run(xp, pk_p, w_p, bias_pad, off)
    return outp[:, :OUT_LEN].reshape(B, OUT_LEN, 1)


# speedup vs baseline: 1.2939x; 1.0398x over previous
import dataclasses
import functools

import jax
import jax.numpy as jnp
from jax import lax
from jax.experimental import pallas as pl
from jax.experimental.pallas import tpu as pltpu
from jax.experimental.pallas import tpu_sc as plsc

B = 16
IN_LEN = 50000
OUT_LEN = 5000
NNZ = 1600000
NCORES = 2
LANES = 16
NE = 8
ELEN = 640
OUT_PAD = NE * ELEN
SBLK = 8192
LSTRIDE = SBLK // LANES
NBLK = -(-NNZ // SBLK)
NNZ_PAD = NBLK * SBLK
CHUNK = 4096
UNROLL = 4


def _body(xp_hbm, pk_hbm, w_hbm, bias_hbm, off_hbm, out_hbm,
          xq0, xq1, acc0, acc1, acc2, acc3, pbuf0, wbuf0, pbuf1, wbuf1,
          offv, bbuf, sem_a, sem_b):
    c_idx = lax.axis_index("c")
    s_idx = lax.axis_index("s")
    wid = s_idx * NCORES + c_idx
    bq = wid % (B // 4)
    e = wid // (B // 4)

    pltpu.sync_copy(xp_hbm.at[2 * bq], xq0)
    pltpu.sync_copy(xp_hbm.at[2 * bq + 1], xq1)
    pltpu.sync_copy(off_hbm, offv)

    iot = lax.iota(jnp.int32, LANES)
    iot_s = iot * LSTRIDE
    ov = offv[...]
    n_lo = jnp.sum(jnp.where(iot == e, ov, 0))
    n_hi = jnp.sum(jnp.where(iot == e + 1, ov, 0))

    c0 = e * ELEN
    pltpu.sync_copy(bias_hbm.at[pl.ds(c0, ELEN)], bbuf)

    @pl.loop(0, ELEN, step=LANES)
    def _init(j):
        bv = bbuf[pl.ds(j, LANES)]
        acc0[pl.ds(j, LANES)] = bv
        acc1[pl.ds(j, LANES)] = bv
        acc2[pl.ds(j, LANES)] = bv
        acc3[pl.ds(j, LANES)] = bv

    lo_al = n_lo & ~(SBLK - 1)
    hi_end = (n_hi + SBLK - 1) & ~(SBLK - 1)
    nchunks = (hi_end - lo_al) // CHUNK

    def copies(k, pb, wb, sem):
        base = pl.multiple_of(
            jnp.minimum(lo_al + k * CHUNK, NNZ_PAD - CHUNK), CHUNK)
        return (
            pltpu.make_async_copy(pk_hbm.at[pl.ds(base, CHUNK)], pb, sem),
            pltpu.make_async_copy(w_hbm.at[pl.ds(base, CHUNK)], wb, sem),
        )

    def issue(k, pb, wb, sem):
        for c in copies(k, pb, wb, sem):
            c.start()

    def drain(k, pb, wb, sem):
        for c in copies(k, pb, wb, sem):
            c.wait()

    def macs(pb, wb, j, m):
        sl = pl.ds(j, LANES)
        pk = pb[sl]
        r = pk & 0xFFFF
        cv = jnp.right_shift(pk, 16)
        wv = wb[sl]
        x01 = plsc.load_gather(xq0, [r])
        x23 = plsc.load_gather(xq1, [r])
        xe0 = plsc.bitcast(jnp.left_shift(x01, 16), jnp.float32)
        xe1 = plsc.bitcast(x01 & jnp.int32(-65536), jnp.float32)
        xe2 = plsc.bitcast(jnp.left_shift(x23, 16), jnp.float32)
        xe3 = plsc.bitcast(x23 & jnp.int32(-65536), jnp.float32)
        plsc.addupdate_scatter(acc0, [cv], xe0 * wv, mask=m)
        plsc.addupdate_scatter(acc1, [cv], xe1 * wv, mask=m)
        plsc.addupdate_scatter(acc2, [cv], xe2 * wv, mask=m)
        plsc.addupdate_scatter(acc3, [cv], xe3 * wv, mask=m)

    def compute(k, pb, wb):
        start = lo_al + k * CHUNK
        blk0 = start & ~(SBLK - 1)
        interior = (blk0 >= n_lo) & (blk0 + SBLK <= n_hi)

        @pl.when(interior)
        def _fast():
            @plsc.parallel_loop(0, CHUNK, step=LANES, unroll=UNROLL)
            def _grp(j):
                macs(pb, wb, j, None)

        @pl.when(jnp.logical_not(interior))
        def _masked():
            gbase = blk0 + ((start & (SBLK - 1)) >> 4)

            @plsc.parallel_loop(0, CHUNK, step=LANES, unroll=UNROLL)
            def _grp(j):
                g = (gbase + (j >> 4)) + iot_s
                m = (g >= n_lo) & (g < n_hi)
                macs(pb, wb, j, m)

    issue(0, pbuf0, wbuf0, sem_a)
    npairs = (nchunks + 1) // 2

    def pair(p, carry):
        k0 = 2 * p
        drain(k0, pbuf0, wbuf0, sem_a)
        issue(k0 + 1, pbuf1, wbuf1, sem_b)
        compute(k0, pbuf0, wbuf0)
        drain(k0 + 1, pbuf1, wbuf1, sem_b)
        issue(k0 + 2, pbuf0, wbuf0, sem_a)
        compute(k0 + 1, pbuf1, wbuf1)
        return carry

    lax.fori_loop(0, npairs, pair, 0)
    drain(2 * npairs, pbuf0, wbuf0, sem_a)

    pltpu.sync_copy(acc0, out_hbm.at[4 * bq + 0, pl.ds(c0, ELEN)])
    pltpu.sync_copy(acc1, out_hbm.at[4 * bq + 1, pl.ds(c0, ELEN)])
    pltpu.sync_copy(acc2, out_hbm.at[4 * bq + 2, pl.ds(c0, ELEN)])
    pltpu.sync_copy(acc3, out_hbm.at[4 * bq + 3, pl.ds(c0, ELEN)])


def _stripe(a):
    a = jnp.pad(a, (0, NNZ_PAD - NNZ))
    return a.reshape(NBLK, LANES, LSTRIDE).transpose(0, 2, 1).reshape(-1)


@jax.jit
def kernel(x, mask_rows, mask_cols, kernel, bias):
    x2 = x.reshape(B, IN_LEN)
    xbf = lax.bitcast_convert_type(
        x2.astype(jnp.bfloat16), jnp.uint16).astype(jnp.uint32)
    xp = lax.bitcast_convert_type(
        xbf[0::2] | (xbf[1::2] << 16), jnp.int32)

    bias_pad = jnp.pad(bias[:, 0], (0, OUT_PAD - OUT_LEN))
    eb = jnp.searchsorted(
        mask_cols, jnp.arange(ELEN, OUT_LEN, ELEN, dtype=jnp.int32)
    ).astype(jnp.int32)
    off = jnp.zeros((LANES,), jnp.int32)
    off = off.at[1:NE].set(eb)
    off = off.at[NE:].set(NNZ)

    local_col = mask_cols - (mask_cols // ELEN) * ELEN
    packed = jnp.bitwise_or(jnp.left_shift(local_col, 16), mask_rows)
    pk_p = _stripe(packed)
    w_p = _stripe(kernel)

    mesh = plsc.VectorSubcoreMesh(core_axis_name="c", subcore_axis_name="s")
    cp = pltpu.CompilerParams()
    if "needs_layout_passes" in pltpu.CompilerParams.__dataclass_fields__:
        cp = dataclasses.replace(cp, needs_layout_passes=False)
    run = functools.partial(
        pl.kernel,
        compiler_params=cp,
        out_type=jax.ShapeDtypeStruct((B, OUT_PAD), jnp.float32),
        mesh=mesh,
        scratch_types=[
            pltpu.VMEM((IN_LEN,), jnp.int32),
            pltpu.VMEM((IN_LEN,), jnp.int32),
            pltpu.VMEM((ELEN,), jnp.float32),
            pltpu.VMEM((ELEN,), jnp.float32),
            pltpu.VMEM((ELEN,), jnp.float32),
            pltpu.VMEM((ELEN,), jnp.float32),
            pltpu.VMEM((CHUNK,), jnp.int32),
            pltpu.VMEM((CHUNK,), jnp.float32),
            pltpu.VMEM((CHUNK,), jnp.int32),
            pltpu.VMEM((CHUNK,), jnp.float32),
            pltpu.VMEM((LANES,), jnp.int32),
            pltpu.VMEM((ELEN,), jnp.float32),
            pltpu.SemaphoreType.DMA,
            pltpu.SemaphoreType.DMA,
        ],
    )(_body)
    outp = run(xp, pk_p, w_p, bias_pad, off)
    return outp[:, :OUT_LEN].reshape(B, OUT_LEN, 1)
